# baseline (speedup 1.0000x reference)
import jax
import jax.numpy as jnp
from jax.experimental import pallas as pl
from jax.experimental.pallas import tpu as pltpu


def _superpose_block(w_ref, k_ref, o_ref):
    w = w_ref[...]
    m = jnp.max(w, axis=-1, keepdims=True)
    e = jnp.exp(w - m)
    probs = e / jnp.sum(e, axis=-1, keepdims=True)
    o_ref[...] = jax.lax.dot_general(
        probs, k_ref[...],
        dimension_numbers=(((1,), (0,)), ((), ())),
        preferred_element_type=jnp.float32)


def kernel(weights, kernel):
    E, H, W = kernel.shape
    B = weights.shape[0]

    BI = 64
    grid = (H // BI,)
    out = pl.pallas_call(
        _superpose_block,
        grid=grid,
        in_specs=[
            pl.BlockSpec((B, E), lambda i: (0, 0)),
            pl.BlockSpec((E, BI, W), lambda i: (0, i, 0)),
        ],
        out_specs=pl.BlockSpec((B, BI, W), lambda i: (0, i, 0)),
        out_shape=jax.ShapeDtypeStruct((B, H, W), jnp.float32),
        compiler_params=pltpu.CompilerParams(
            dimension_semantics=("parallel",),
            vmem_limit_bytes=100 * 1024 * 1024),
    )(weights, kernel)
    return out


# speedup vs baseline: 1.1252x; 1.1252x over previous
import jax
import jax.numpy as jnp
from jax.experimental import pallas as pl
from jax.experimental.pallas import tpu as pltpu

NBUF = 4
BI = 32


def _body(w_ref, k_hbm, o_hbm, kbuf, obuf, in_sems, out_sems):
    E = k_hbm.shape[0]
    H = k_hbm.shape[1]
    W = k_hbm.shape[2]
    B = w_ref.shape[0]
    nsteps = H // BI

    w = w_ref[...]
    m = jnp.max(w, axis=-1, keepdims=True)
    e = jnp.exp(w - m)
    probs = e / jnp.sum(e, axis=-1, keepdims=True)

    def in_copy(step):
        slot = jax.lax.rem(step, NBUF)
        return pltpu.make_async_copy(
            k_hbm.at[:, pl.ds(step * BI, BI), :], kbuf.at[slot],
            in_sems.at[slot])

    def out_copy(step):
        slot = jax.lax.rem(step, NBUF)
        return pltpu.make_async_copy(
            obuf.at[slot], o_hbm.at[:, pl.ds(step * BI, BI), :],
            out_sems.at[slot])

    for s in range(NBUF):
        in_copy(s).start()

    def step_fn(i, _):
        slot = jax.lax.rem(i, NBUF)
        in_copy(i).wait()

        @pl.when(i >= NBUF)
        def _():
            out_copy(i - NBUF).wait()

        obuf[slot] = jax.lax.dot_general(
            probs, kbuf[slot],
            dimension_numbers=(((1,), (0,)), ((), ())),
            preferred_element_type=jnp.float32)
        out_copy(i).start()

        @pl.when(i + NBUF < nsteps)
        def _():
            in_copy(i + NBUF).start()
        return 0

    jax.lax.fori_loop(0, nsteps, step_fn, 0)

    for s in range(NBUF):
        out_copy(nsteps - NBUF + s).wait()


def kernel(weights, kernel):
    E, H, W = kernel.shape
    B = weights.shape[0]
    return pl.pallas_call(
        _body,
        in_specs=[
            pl.BlockSpec((B, E), lambda: (0, 0)),
            pl.BlockSpec(memory_space=pltpu.MemorySpace.HBM),
        ],
        out_specs=pl.BlockSpec(memory_space=pltpu.MemorySpace.HBM),
        out_shape=jax.ShapeDtypeStruct((B, H, W), jnp.float32),
        scratch_shapes=[
            pltpu.VMEM((NBUF, E, BI, W), jnp.float32),
            pltpu.VMEM((NBUF, B, BI, W), jnp.float32),
            pltpu.SemaphoreType.DMA((NBUF,)),
            pltpu.SemaphoreType.DMA((NBUF,)),
        ],
        compiler_params=pltpu.CompilerParams(
            vmem_limit_bytes=100 * 1024 * 1024),
    )(weights, kernel)


# speedup vs baseline: 1.1369x; 1.0104x over previous
import jax
import jax.numpy as jnp
from jax.experimental import pallas as pl
from jax.experimental.pallas import tpu as pltpu

NBUF = 6
BI = 16


def _body(w_ref, k_hbm, o_hbm, kbuf, obuf, in_sems, out_sems):
    E = k_hbm.shape[0]
    H = k_hbm.shape[1]
    W = k_hbm.shape[2]
    B = w_ref.shape[0]
    nsteps = H // BI

    w = w_ref[...]
    m = jnp.max(w, axis=-1, keepdims=True)
    e = jnp.exp(w - m)
    probs = e / jnp.sum(e, axis=-1, keepdims=True)

    def in_copy(step):
        slot = jax.lax.rem(step, NBUF)
        return pltpu.make_async_copy(
            k_hbm.at[:, pl.ds(step * BI, BI), :], kbuf.at[slot],
            in_sems.at[slot])

    def out_copy(step):
        slot = jax.lax.rem(step, NBUF)
        return pltpu.make_async_copy(
            obuf.at[slot], o_hbm.at[:, pl.ds(step * BI, BI), :],
            out_sems.at[slot])

    for s in range(NBUF):
        in_copy(s).start()

    def step_fn(i, _):
        slot = jax.lax.rem(i, NBUF)
        in_copy(i).wait()

        @pl.when(i >= NBUF)
        def _():
            out_copy(i - NBUF).wait()

        obuf[slot] = jax.lax.dot_general(
            probs, kbuf[slot],
            dimension_numbers=(((1,), (0,)), ((), ())),
            preferred_element_type=jnp.float32)
        out_copy(i).start()

        @pl.when(i + NBUF < nsteps)
        def _():
            in_copy(i + NBUF).start()
        return 0

    jax.lax.fori_loop(0, nsteps, step_fn, 0)

    for s in range(NBUF):
        out_copy(nsteps - NBUF + s).wait()


def kernel(weights, kernel):
    E, H, W = kernel.shape
    B = weights.shape[0]
    return pl.pallas_call(
        _body,
        in_specs=[
            pl.BlockSpec((B, E), lambda: (0, 0)),
            pl.BlockSpec(memory_space=pltpu.MemorySpace.HBM),
        ],
        out_specs=pl.BlockSpec(memory_space=pltpu.MemorySpace.HBM),
        out_shape=jax.ShapeDtypeStruct((B, H, W), jnp.float32),
        scratch_shapes=[
            pltpu.VMEM((NBUF, E, BI, W), jnp.float32),
            pltpu.VMEM((NBUF, B, BI, W), jnp.float32),
            pltpu.SemaphoreType.DMA((NBUF,)),
            pltpu.SemaphoreType.DMA((NBUF,)),
        ],
        compiler_params=pltpu.CompilerParams(
            vmem_limit_bytes=100 * 1024 * 1024),
    )(weights, kernel)
